# baseline (speedup 1.0000x reference)
import functools
import jax
import jax.numpy as jnp
from jax import lax
from jax.experimental import pallas as pl
from jax.experimental.pallas import tpu as pltpu
from jax.experimental.pallas import tpu_sc as plsc

BATCH = 16384
D = 32
NC = 2
NS = 16
LANES = 16
NW = NC * NS
BPW = BATCH // NW
CHUNK = 128
NCHUNK = BPW // CHUNK
NG = BPW // LANES


def _mf_body(uid_hbm, iid_hbm, p_hbm, q_hbm, ub_hbm, ib_hbm, out_hbm,
             uidx_v, iidx_v, urows_v, irows_v, ub_v, ib_v, out_v, sem):
    wid = lax.axis_index("s") * NC + lax.axis_index("c")

    pltpu.sync_copy(uid_hbm.at[pl.ds(wid * NCHUNK, NCHUNK)], uidx_v)
    pltpu.sync_copy(iid_hbm.at[pl.ds(wid * NCHUNK, NCHUNK)], iidx_v)

    copies = []
    for k in range(NCHUNK):
        sl = pl.ds(k * CHUNK, CHUNK)
        copies.append(pltpu.async_copy(p_hbm.at[uidx_v.at[k]], urows_v.at[sl], sem))
        copies.append(pltpu.async_copy(q_hbm.at[iidx_v.at[k]], irows_v.at[sl], sem))
        copies.append(pltpu.async_copy(ub_hbm.at[uidx_v.at[k]], ub_v.at[sl], sem))
        copies.append(pltpu.async_copy(ib_hbm.at[iidx_v.at[k]], ib_v.at[sl], sem))
    for c in copies:
        c.wait()

    iota = lax.iota(jnp.int32, LANES)

    def group(g, carry):
        rbase = g * LANES
        row_idx = rbase + iota
        acc = jnp.zeros((LANES,), jnp.float32)
        for c in range(D):
            col = jnp.bitwise_and(iota + c, D - 1)
            uv = plsc.load_gather(urows_v, [row_idx, col])
            iv = plsc.load_gather(irows_v, [row_idx, col])
            acc = acc + uv * iv
        out_v[pl.ds(rbase, LANES)] = (
            acc + ub_v[pl.ds(rbase, LANES)] + ib_v[pl.ds(rbase, LANES)])
        return carry

    lax.fori_loop(0, NG, group, 0)

    pltpu.sync_copy(out_v, out_hbm.at[pl.ds(wid * BPW, BPW)])


@jax.jit
def _mf_sc(uidx, iidx, p, q, ub, ib):
    mesh = plsc.VectorSubcoreMesh(core_axis_name="c", subcore_axis_name="s")
    f = pl.kernel(
        _mf_body,
        out_type=jax.ShapeDtypeStruct((BATCH,), jnp.float32),
        mesh=mesh,
        compiler_params=pltpu.CompilerParams(
            needs_layout_passes=False, use_tc_tiling_on_sc=False),
        scratch_types=[
            pltpu.VMEM((NCHUNK, CHUNK), jnp.int32),
            pltpu.VMEM((NCHUNK, CHUNK), jnp.int32),
            pltpu.VMEM((BPW, D), jnp.float32),
            pltpu.VMEM((BPW, D), jnp.float32),
            pltpu.VMEM((BPW,), jnp.float32),
            pltpu.VMEM((BPW,), jnp.float32),
            pltpu.VMEM((BPW,), jnp.float32),
            pltpu.SemaphoreType.DMA,
        ],
    )
    return f(uidx, iidx, p, q, ub, ib)


def kernel(user_id, item_id, p, q, user_bias, item_bias):
    uidx = (user_id - 1).reshape(NW * NCHUNK, CHUNK)
    iidx = (item_id - 1).reshape(NW * NCHUNK, CHUNK)
    out = _mf_sc(uidx, iidx, p, q,
                 user_bias.reshape(-1), item_bias.reshape(-1))
    return out.reshape(BATCH, 1)


# speedup vs baseline: 2.6226x; 2.6226x over previous
import jax
import jax.numpy as jnp
from jax import lax
from jax.experimental import pallas as pl
from jax.experimental.pallas import tpu as pltpu
from jax.experimental.pallas import tpu_sc as plsc

BATCH = 16384
D = 32
NC = 2
NS = 16
LANES = 16
NW = NC * NS
BPW = BATCH // NW
CHUNK = 128
NCHUNK = BPW // CHUNK
NR = 8
NGRP = BPW // LANES


def _mf_body(uid_hbm, iid_hbm, pt_hbm, qt_hbm, ub_hbm, ib_hbm, out_hbm,
             uc_v, ic_v, uring, iring, ubch, ibch, out_v, sem):
    wid = lax.axis_index("s") * NC + lax.axis_index("c")

    pltpu.sync_copy(uid_hbm.at[wid], uc_v)
    pltpu.sync_copy(iid_hbm.at[wid], ic_v)

    bcopies = []
    for k in range(NCHUNK):
        bcopies.append(pltpu.async_copy(ub_hbm.at[uc_v.at[k]], ubch.at[k], sem))
        bcopies.append(pltpu.async_copy(ib_hbm.at[ic_v.at[k]], ibch.at[k], sem))
    for c in bcopies:
        c.wait()

    iota = lax.iota(jnp.int32, LANES)
    cg_lo = lax.shift_right_logical(iota, 3)
    cg_hi = cg_lo + 2
    cl16 = jnp.bitwise_and(iota, 7)

    def group_body(g, carry):
        row = lax.shift_right_logical(g, 3)
        colb = jnp.bitwise_and(g, 7) * LANES
        sl = pl.ds(colb, LANES)
        uv16 = uc_v[row, sl]
        iv16 = ic_v[row, sl]
        acc = jnp.zeros((LANES,), jnp.float32)

        for h in range(2):
            fired = []
            for j in range(NR):
                uid = uv16[NR * h + j]
                iid = iv16[NR * h + j]
                ut = lax.shift_right_logical(uid, 7) * 128
                it = lax.shift_right_logical(iid, 7) * 128
                fired.append(pltpu.async_copy(
                    pt_hbm.at[:, :, pl.ds(ut, 128)], uring.at[j], sem))
                fired.append(pltpu.async_copy(
                    qt_hbm.at[:, :, pl.ds(it, 128)], iring.at[j], sem))
            for c in fired:
                c.wait()

            for j in range(NR):
                uid = uv16[NR * h + j]
                iid = iv16[NR * h + j]
                uil = jnp.broadcast_to(jnp.bitwise_and(uid, 127), (LANES,))
                iil = jnp.broadcast_to(jnp.bitwise_and(iid, 127), (LANES,))
                u_lo = plsc.load_gather(uring.at[j], [cg_lo, cl16, uil])
                u_hi = plsc.load_gather(uring.at[j], [cg_hi, cl16, uil])
                i_lo = plsc.load_gather(iring.at[j], [cg_lo, cl16, iil])
                i_hi = plsc.load_gather(iring.at[j], [cg_hi, cl16, iil])
                prod = u_lo * i_lo + u_hi * i_hi
                acc = jnp.where(iota == (NR * h + j),
                                jnp.broadcast_to(jnp.sum(prod), (LANES,)), acc)

        out_v[row, sl] = acc + ubch[row, sl] + ibch[row, sl]
        return carry

    lax.fori_loop(0, NGRP, group_body, 0)

    pltpu.sync_copy(out_v, out_hbm.at[wid])


@jax.jit
def _mf_sc(uidx, iidx, pt, qt, ub, ib):
    mesh = plsc.VectorSubcoreMesh(core_axis_name="c", subcore_axis_name="s")
    f = pl.kernel(
        _mf_body,
        out_type=jax.ShapeDtypeStruct((NW, NCHUNK, CHUNK), jnp.float32),
        mesh=mesh,
        compiler_params=pltpu.CompilerParams(
            needs_layout_passes=False, use_tc_tiling_on_sc=True),
        scratch_types=[
            pltpu.VMEM((NCHUNK, CHUNK), jnp.int32),
            pltpu.VMEM((NCHUNK, CHUNK), jnp.int32),
            pltpu.VMEM((NR, 4, 8, 128), jnp.float32),
            pltpu.VMEM((NR, 4, 8, 128), jnp.float32),
            pltpu.VMEM((NCHUNK, CHUNK), jnp.float32),
            pltpu.VMEM((NCHUNK, CHUNK), jnp.float32),
            pltpu.VMEM((NCHUNK, CHUNK), jnp.float32),
            pltpu.SemaphoreType.DMA,
        ],
    )
    return f(uidx, iidx, pt, qt, ub, ib)


def kernel(user_id, item_id, p, q, user_bias, item_bias):
    uidx = (user_id - 1).reshape(NW, NCHUNK, CHUNK)
    iidx = (item_id - 1).reshape(NW, NCHUNK, CHUNK)
    pt = jnp.transpose(p).reshape(4, 8, 1000000)
    qt = jnp.transpose(q).reshape(4, 8, 1000000)
    ub = jnp.sum(user_bias, axis=1)
    ib = jnp.sum(item_bias, axis=1)
    out = _mf_sc(uidx, iidx, pt, qt, ub, ib)
    return out.reshape(BATCH, 1)
